# baseline (speedup 1.0000x reference)
import functools

import jax
import jax.numpy as jnp
from jax import lax
from jax.experimental import pallas as pl
from jax.experimental.pallas import tpu as pltpu
from jax.experimental.pallas import tpu_sc as plsc

HIDDEN = 1024
ROWS = 4 * 8192
SC_ROWS = 21 * 1024
TC_ROWS = ROWS - SC_ROWS

NC, NS, LANES = 2, 16, 16
NW = NC * NS
ROWS_PER_W = SC_ROWS // NW
CHUNK = 8
N_CHUNKS = ROWS_PER_W // CHUNK
NBUF = 4
VECS_PER_ROW = HIDDEN // LANES

BR = 128
N_TC_BLOCKS = TC_ROWS // BR


def _sc_gather_add(pe, ids, x):
    mesh = plsc.VectorSubcoreMesh(core_axis_name="c", subcore_axis_name="s")

    @functools.partial(
        pl.kernel,
        mesh=mesh,
        out_type=jax.ShapeDtypeStruct((SC_ROWS, HIDDEN), jnp.float32),
        scratch_types=[
            pltpu.VMEM((N_CHUNKS, CHUNK), jnp.int32),
            [pltpu.VMEM((CHUNK, HIDDEN), jnp.float32) for _ in range(NBUF)],
            [pltpu.VMEM((CHUNK, HIDDEN), jnp.float32) for _ in range(NBUF)],
            [pltpu.VMEM((CHUNK, HIDDEN), jnp.float32) for _ in range(NBUF)],
            [pltpu.SemaphoreType.DMA for _ in range(3 * NBUF)],
        ],
    )
    def k(pe_hbm, ids_hbm, x_hbm, out_hbm, idx_all, pe_v, x_v, o_v, sems):
        wid = lax.axis_index("s") * NC + lax.axis_index("c")
        w_base = wid * ROWS_PER_W
        gsem = sems[0:NBUF]
        xsem = sems[NBUF:2 * NBUF]
        osem = sems[2 * NBUF:3 * NBUF]

        pltpu.sync_copy(ids_hbm.at[wid], idx_all)

        def start_in(ci, b):
            pltpu.async_copy(pe_hbm.at[idx_all.at[ci]], pe_v[b], gsem[b])
            pltpu.async_copy(x_hbm.at[pl.ds(w_base + ci * CHUNK, CHUNK)],
                             x_v[b], xsem[b])

        for p in range(NBUF):
            start_in(p, p)

        @pl.loop(0, N_CHUNKS, step=NBUF)
        def chunk_group(ci0):
            for b in range(NBUF):
                ci = ci0 + b
                base = w_base + ci * CHUNK
                pltpu.make_async_copy(pe_hbm.at[idx_all.at[ci]],
                                      pe_v[b], gsem[b]).wait()
                pltpu.make_async_copy(x_hbm.at[pl.ds(base, CHUNK)],
                                      x_v[b], xsem[b]).wait()

                @pl.when(ci >= NBUF)
                def _():
                    pltpu.make_async_copy(
                        o_v[b], out_hbm.at[pl.ds(base, CHUNK)], osem[b]
                    ).wait()

                for r in range(CHUNK):
                    @plsc.parallel_loop(0, VECS_PER_ROW, unroll=8)
                    def add_vec(v):
                        sl = pl.ds(v * LANES, LANES)
                        o_v[b][r, sl] = x_v[b][r, sl] + pe_v[b][r, sl]

                pltpu.async_copy(o_v[b], out_hbm.at[pl.ds(base, CHUNK)],
                                 osem[b])

                @pl.when(ci + NBUF < N_CHUNKS)
                def _():
                    start_in(ci + NBUF, b)

        for b in range(NBUF):
            ci = N_CHUNKS - NBUF + b
            pltpu.make_async_copy(
                o_v[b],
                out_hbm.at[pl.ds(w_base + ci * CHUNK, CHUNK)],
                osem[b],
            ).wait()

    return k(pe, ids, x)


def _tc_kernel(ids_ref, x_ref, pe_hbm, o_ref, gbuf, sems):
    i = pl.program_id(0)
    nb = pl.num_programs(0)

    def issue(blk, buf):
        def body(r, _):
            idx = ids_ref[blk * BR + r]
            pltpu.make_async_copy(
                pe_hbm.at[idx], gbuf.at[buf, r], sems.at[buf]
            ).start()
            return 0
        lax.fori_loop(0, BR, body, 0)

    @pl.when(i == 0)
    def _():
        issue(0, 0)

    @pl.when(i + 1 < nb)
    def _():
        issue(i + 1, (i + 1) % 2)

    par = i % 2

    def drain(r, _):
        pltpu.make_async_copy(
            pe_hbm.at[0], gbuf.at[par, 0], sems.at[par]
        ).wait()
        return 0
    lax.fori_loop(0, BR, drain, 0)

    o_ref[...] = x_ref[...] + gbuf[par]


def _tc_gather_add(pe, ids, x):
    grid_spec = pltpu.PrefetchScalarGridSpec(
        num_scalar_prefetch=1,
        grid=(N_TC_BLOCKS,),
        in_specs=[
            pl.BlockSpec((BR, HIDDEN), lambda i, ids: (i, 0)),
            pl.BlockSpec(memory_space=pltpu.MemorySpace.HBM),
        ],
        out_specs=pl.BlockSpec((BR, HIDDEN), lambda i, ids: (i, 0)),
        scratch_shapes=[
            pltpu.VMEM((2, BR, HIDDEN), jnp.float32),
            pltpu.SemaphoreType.DMA((2,)),
        ],
    )
    return pl.pallas_call(
        _tc_kernel,
        grid_spec=grid_spec,
        out_shape=jax.ShapeDtypeStruct((TC_ROWS, HIDDEN), jnp.float32),
    )(ids, x, pe)


def kernel(x, temporal_ids, pe):
    b, s, h = x.shape
    x2 = x.reshape(b * s, h)
    ids = temporal_ids.reshape(-1).astype(jnp.int32)
    ids_sc = ids[:SC_ROWS].reshape(NW, N_CHUNKS, CHUNK)
    out_sc = _sc_gather_add(pe, ids_sc, x2[:SC_ROWS])
    out_tc = _tc_gather_add(pe, ids[SC_ROWS:], x2[SC_ROWS:])
    return jnp.concatenate([out_sc, out_tc], axis=0).reshape(b, s, h)


# speedup vs baseline: 2.6055x; 2.6055x over previous
import functools

import jax
import jax.numpy as jnp
from jax import lax
from jax.experimental import pallas as pl
from jax.experimental.pallas import tpu as pltpu
from jax.experimental.pallas import tpu_sc as plsc

HIDDEN = 1024
ROWS = 4 * 8192
NC, NS, LANES = 2, 16, 16
NW = NC * NS
ROWS_PER_W = ROWS // NW
CHUNK = 8
N_CHUNKS = ROWS_PER_W // CHUNK
VECS_PER_ROW = HIDDEN // LANES


def _sc_gather_add(pe, ids, x):
    mesh = plsc.VectorSubcoreMesh(core_axis_name="c", subcore_axis_name="s")

    @functools.partial(
        pl.kernel,
        mesh=mesh,
        out_type=jax.ShapeDtypeStruct((ROWS, HIDDEN), jnp.float32),
        scratch_types=[
            pltpu.VMEM((N_CHUNKS, CHUNK), jnp.int32),
            [pltpu.VMEM((CHUNK, HIDDEN), jnp.float32) for _ in range(4)],
            [pltpu.VMEM((CHUNK, HIDDEN), jnp.float32) for _ in range(4)],
            [pltpu.VMEM((CHUNK, HIDDEN), jnp.float32) for _ in range(4)],
            [pltpu.SemaphoreType.DMA for _ in range(12)],
        ],
    )
    def k(pe_hbm, ids_hbm, x_hbm, out_hbm, idx_all, pe_v, x_v, o_v, sems):
        wid = lax.axis_index("s") * NC + lax.axis_index("c")
        w_base = wid * ROWS_PER_W
        gsem, xsem, osem = sems[0:4], sems[4:8], sems[8:12]

        pltpu.sync_copy(ids_hbm.at[wid], idx_all)

        def start_in(ci, b):
            pltpu.async_copy(pe_hbm.at[idx_all.at[ci]], pe_v[b], gsem[b])
            pltpu.async_copy(x_hbm.at[pl.ds(w_base + ci * CHUNK, CHUNK)],
                             x_v[b], xsem[b])

        for p in range(4):
            start_in(p, p)

        @pl.loop(0, N_CHUNKS, step=4)
        def chunk_group(ci0):
            for b in range(4):
                ci = ci0 + b
                base = w_base + ci * CHUNK
                pltpu.make_async_copy(pe_hbm.at[idx_all.at[ci]],
                                      pe_v[b], gsem[b]).wait()
                pltpu.make_async_copy(x_hbm.at[pl.ds(base, CHUNK)],
                                      x_v[b], xsem[b]).wait()

                @pl.when(ci >= 4)
                def _():
                    pltpu.make_async_copy(
                        o_v[b], out_hbm.at[pl.ds(base, CHUNK)], osem[b]
                    ).wait()

                for r in range(CHUNK):
                    @plsc.parallel_loop(0, VECS_PER_ROW, unroll=8)
                    def add_vec(v):
                        sl = pl.ds(v * LANES, LANES)
                        o_v[b][r, sl] = x_v[b][r, sl] + pe_v[b][r, sl]

                pltpu.async_copy(o_v[b], out_hbm.at[pl.ds(base, CHUNK)],
                                 osem[b])

                @pl.when(ci + 4 < N_CHUNKS)
                def _():
                    start_in(ci + 4, b)

        for b in range(4):
            ci = N_CHUNKS - 4 + b
            pltpu.make_async_copy(
                o_v[b],
                out_hbm.at[pl.ds(w_base + ci * CHUNK, CHUNK)],
                osem[b],
            ).wait()

    return k(pe, ids, x)


def kernel(x, temporal_ids, pe):
    b, s, h = x.shape
    x2 = x.reshape(b * s, h)
    ids = temporal_ids.reshape(NW, N_CHUNKS, CHUNK).astype(jnp.int32)
    out = _sc_gather_add(pe, ids, x2)
    return out.reshape(b, s, h)
